# Initial kernel scaffold; baseline (speedup 1.0000x reference)
import jax
import jax.numpy as jnp
from jax.experimental import pallas as pl


def kernel(input, edge_list, W, a):
    raise NotImplementedError("write your pallas kernel here")


# speedup vs baseline: 11.1362x; 11.1362x over previous
import functools

import jax
import jax.numpy as jnp
from jax import lax
from jax.experimental import pallas as pl
from jax.experimental.pallas import tpu as pltpu
from jax.experimental.pallas import tpu_sc as plsc

N = 10000
E = 320000
D = 128

NC = 2
NS = 16
L = 16
NW = NC * NS
EPW = E // NW
G = EPW // L
DC = D + L
NP = 10240
RPT = NP // NS
ZR = 16
CH = 2000
NCH = EPW // CH


def _prep(x, w, a2d):
    B = 2000
    grid = N // B

    def body(x_ref, w_ref, a_ref, h_ref, s_ref, t_ref):
        h = jnp.dot(x_ref[...], w_ref[...], preferred_element_type=jnp.float32)
        h_ref[...] = h
        a1 = a_ref[0, :]
        a2 = a_ref[1, :]
        s_ref[...] = jnp.dot(h, a1, preferred_element_type=jnp.float32).reshape(1, 1, B)
        t_ref[...] = jnp.dot(h, a2, preferred_element_type=jnp.float32).reshape(1, 1, B)

    h, s3, t3 = pl.pallas_call(
        body,
        grid=(grid,),
        in_specs=[
            pl.BlockSpec((B, D), lambda i: (i, 0)),
            pl.BlockSpec((D, D), lambda i: (0, 0)),
            pl.BlockSpec((2, D), lambda i: (0, 0)),
        ],
        out_specs=[
            pl.BlockSpec((B, D), lambda i: (i, 0)),
            pl.BlockSpec((1, 1, B), lambda i: (i, 0, 0)),
            pl.BlockSpec((1, 1, B), lambda i: (i, 0, 0)),
        ],
        out_shape=[
            jax.ShapeDtypeStruct((N, D), jnp.float32),
            jax.ShapeDtypeStruct((grid, 1, B), jnp.float32),
            jax.ShapeDtypeStruct((grid, 1, B), jnp.float32),
        ],
    )(x, w, a2d)
    return h, s3.reshape(N), t3.reshape(N)


def _sc_edge(src3, dst3, s, t, h):
    mesh = plsc.VectorSubcoreMesh(core_axis_name="c", subcore_axis_name="s")

    @functools.partial(
        pl.kernel,
        out_type=(
            jax.ShapeDtypeStruct((NC, NP, D), jnp.float32),
            jax.ShapeDtypeStruct((NW, N), jnp.float32),
        ),
        mesh=mesh,
        scratch_types=[
            pltpu.VMEM((CH,), jnp.int32),
            pltpu.VMEM((CH,), jnp.int32),
            pltpu.VMEM((N,), jnp.float32),
            pltpu.VMEM((N,), jnp.float32),
            pltpu.VMEM((N,), jnp.float32),
            pltpu.VMEM((L, D), jnp.float32),
            pltpu.VMEM((8, L), jnp.int32),
            pltpu.VMEM((ZR, D), jnp.float32),
            pltpu.VMEM_SHARED((NP, D), jnp.float32),
            pltpu.SemaphoreType.DMA,
            pltpu.SemaphoreType.DMA,
        ],
        compiler_params=pltpu.CompilerParams(needs_layout_passes=False),
    )
    def k(src_h, dst_h, s_h, t_h, h_h, num_h, den_h,
          srcs, dsts, sv, tv, den, rows, idxb, zbuf, acc, sem, sem2):
        cid = lax.axis_index("c")
        sid = lax.axis_index("s")
        wid = sid * NC + cid

        pltpu.sync_copy(s_h, sv)
        pltpu.sync_copy(t_h, tv)

        zeros16 = jnp.zeros((L,), jnp.float32)

        def zden(i, carry):
            den[pl.ds(i * L, L)] = zeros16
            return carry

        lax.fori_loop(0, N // L, zden, 0)

        for r in range(ZR):
            for c in range(D // L):
                zbuf[r, pl.ds(c * L, L)] = zeros16
        for b in range(RPT // ZR):
            pltpu.sync_copy(zbuf, acc.at[pl.ds(sid * RPT + b * ZR, ZR)])
        plsc.subcore_barrier()

        lanes = lax.iota(jnp.int32, L)

        def body(g, carry):
            src16 = srcs[pl.ds(g * L, L)]
            dst16 = dsts[pl.ds(g * L, L)]
            svv = plsc.load_gather(sv, [src16])
            tvv = plsc.load_gather(tv, [dst16])
            x = svv + tvv
            e = jnp.where(x > 0.0, x, 0.2 * x)
            p = jnp.exp(e)
            idxb[0, :] = src16
            pltpu.async_copy(h_h.at[dst16], rows, sem).wait()
            for r in range(L):
                pr = p[r]
                for c in range(D // L):
                    rows[r, pl.ds(c * L, L)] = rows[r, pl.ds(c * L, L)] * pr
            pltpu.sync_copy(rows, acc.at[idxb.at[0]], add=True)
            ks, vs = plsc.sort_key_val(src16, p)
            knext = ks.at[jnp.minimum(lanes + 1, L - 1)].get(
                mode="promise_in_bounds")
            ends = (ks != knext) | (lanes == L - 1)
            csum = plsc.cumsum(vs)
            q = plsc.cummax(jnp.where(ends, lanes, -1))
            qprev = q.at[jnp.maximum(lanes - 1, 0)].get(
                mode="promise_in_bounds")
            pe = jnp.where(lanes == 0, -1, qprev)
            cprev = csum.at[jnp.maximum(pe, 0)].get(
                mode="promise_in_bounds")
            runsum = csum - jnp.where(pe >= 0, cprev, 0.0)
            plsc.addupdate_scatter(den, [ks], runsum, mask=ends)
            return carry

        def chunk(ch, carry):
            pltpu.sync_copy(src_h.at[wid, ch], srcs)
            pltpu.sync_copy(dst_h.at[wid, ch], dsts)
            lax.fori_loop(0, CH // L, body, 0)
            return carry

        lax.fori_loop(0, NCH, chunk, 0)
        pltpu.sync_copy(den, den_h.at[wid])
        plsc.subcore_barrier()
        pltpu.sync_copy(acc.at[pl.ds(sid * RPT, RPT)],
                        num_h.at[cid, pl.ds(sid * RPT, RPT)])

    return k(src3, dst3, s, t, h)


def _finish(num_parts, den_parts):
    B = 2000

    def body(n_ref, d_ref, o_ref):
        nblk = n_ref[...]
        num = nblk[0] + nblk[1]
        den = jnp.sum(d_ref[...], axis=1)
        safe = jnp.where(den == 0.0, 1.0, den)
        r = num / safe[:, None]
        neg = jnp.exp(jnp.minimum(r, 0.0)) - 1.0
        out = jnp.where(r > 0.0, r, neg)
        o_ref[...] = jnp.where(den[:, None] == 0.0, 0.0, out)

    return pl.pallas_call(
        body,
        grid=(N // B,),
        in_specs=[
            pl.BlockSpec((NC, B, D), lambda i: (0, i, 0)),
            pl.BlockSpec((B, NW), lambda i: (i, 0)),
        ],
        out_specs=pl.BlockSpec((B, D), lambda i: (i, 0)),
        out_shape=jax.ShapeDtypeStruct((N, D), jnp.float32),
    )(num_parts, den_parts)


def kernel(input, edge_list, W, a):
    src3 = edge_list[0].reshape(NW, NCH, CH)
    dst3 = edge_list[1].reshape(NW, NCH, CH)
    a2d = a.reshape(2, D)
    h, s, t = _prep(input, W, a2d)
    num_parts, den_parts = _sc_edge(src3, dst3, s, t, h)
    return _finish(num_parts, den_parts.T)


# speedup vs baseline: 32.7454x; 2.9404x over previous
import functools

import jax
import jax.numpy as jnp
from jax import lax
from jax.experimental import pallas as pl
from jax.experimental.pallas import tpu as pltpu
from jax.experimental.pallas import tpu_sc as plsc

N = 10000
E = 320000
D = 128

NC = 2
NS = 16
L = 16
NW = NC * NS
EPW = E // NW
G = EPW // L
DC = D + L
NP = 10240
RPT = NP // NS
CH = 2000
NCH = EPW // CH
GPC = CH // L
NB = 5


def _prep(x, w, a2d):
    B = 2000
    grid = N // B

    def body(x_ref, w_ref, a_ref, h_ref, s_ref, t_ref):
        h = jnp.dot(x_ref[...], w_ref[...], preferred_element_type=jnp.float32)
        h_ref[...] = h
        a1 = a_ref[0, :]
        a2 = a_ref[1, :]
        s_ref[...] = jnp.dot(h, a1, preferred_element_type=jnp.float32).reshape(1, 1, B)
        t_ref[...] = jnp.dot(h, a2, preferred_element_type=jnp.float32).reshape(1, 1, B)

    h, s3, t3 = pl.pallas_call(
        body,
        grid=(grid,),
        in_specs=[
            pl.BlockSpec((B, D), lambda i: (i, 0)),
            pl.BlockSpec((D, D), lambda i: (0, 0)),
            pl.BlockSpec((2, D), lambda i: (0, 0)),
        ],
        out_specs=[
            pl.BlockSpec((B, D), lambda i: (i, 0)),
            pl.BlockSpec((1, 1, B), lambda i: (i, 0, 0)),
            pl.BlockSpec((1, 1, B), lambda i: (i, 0, 0)),
        ],
        out_shape=[
            jax.ShapeDtypeStruct((N, D), jnp.float32),
            jax.ShapeDtypeStruct((grid, 1, B), jnp.float32),
            jax.ShapeDtypeStruct((grid, 1, B), jnp.float32),
        ],
    )(x, w, a2d)
    return h, s3.reshape(N), t3.reshape(N)


def _sc_edge(src3, dst3, s, t, h):
    mesh = plsc.VectorSubcoreMesh(core_axis_name="c", subcore_axis_name="s")

    @functools.partial(
        pl.kernel,
        out_type=(
            jax.ShapeDtypeStruct((NC, NP, D), jnp.float32),
            jax.ShapeDtypeStruct((NW, N), jnp.float32),
        ),
        mesh=mesh,
        scratch_types=[
            pltpu.VMEM((CH,), jnp.int32),
            pltpu.VMEM((CH,), jnp.int32),
            pltpu.VMEM((N,), jnp.float32),
            pltpu.VMEM((N,), jnp.float32),
            pltpu.VMEM((N,), jnp.float32),
            [pltpu.VMEM((L, D), jnp.float32) for _ in range(NB)],
            pltpu.VMEM((8, L), jnp.int32),
            pltpu.VMEM_SHARED((NP, D), jnp.float32),
            [pltpu.SemaphoreType.DMA for _ in range(NB)],
            [pltpu.SemaphoreType.DMA for _ in range(NB)],
        ],
        compiler_params=pltpu.CompilerParams(needs_layout_passes=False),
    )
    def k(src_h, dst_h, s_h, t_h, h_h, num_h, den_h,
          srcs, dsts, sv, tv, den, rows, idxb, acc, gsem, ssem):
        cid = lax.axis_index("c")
        sid = lax.axis_index("s")
        wid = sid * NC + cid

        pltpu.sync_copy(s_h, sv)
        pltpu.sync_copy(t_h, tv)

        zeros16 = jnp.zeros((L,), jnp.float32)

        def zden(i, carry):
            den[pl.ds(i * L, L)] = zeros16
            return carry

        lax.fori_loop(0, N // L, zden, 0)
        for c in range(D // L):
            for r in range(L):
                rows[0][r, pl.ds(c * L, L)] = zeros16

        def zacc(i, carry):
            pltpu.sync_copy(rows[0], acc.at[pl.ds(sid * RPT + i * L, L)])
            return carry

        lax.fori_loop(0, RPT // L, zacc, 0)
        plsc.subcore_barrier()

        lanes = lax.iota(jnp.int32, L)

        def process(m, b):
            src16 = srcs[pl.ds(m * L, L)]
            dst16 = dsts[pl.ds(m * L, L)]
            svv = plsc.load_gather(sv, [src16])
            tvv = plsc.load_gather(tv, [dst16])
            x = svv + tvv
            e = jnp.where(x > 0.0, x, 0.2 * x)
            p = jnp.exp(e)
            pltpu.make_async_copy(h_h.at[dst16], rows[b], gsem[b]).wait()
            for r in range(L):
                pr = p[r]
                for c in range(D // L):
                    rows[b][r, pl.ds(c * L, L)] = rows[b][r, pl.ds(c * L, L)] * pr
            idxb[b, :] = src16
            pltpu.async_copy(rows[b], acc.at[idxb.at[b]], ssem[b], add=True)
            ks, vs = plsc.sort_key_val(src16, p)
            knext = ks.at[jnp.minimum(lanes + 1, L - 1)].get(
                mode="promise_in_bounds")
            ends = (ks != knext) | (lanes == L - 1)
            csum = plsc.cumsum(vs)
            q = plsc.cummax(jnp.where(ends, lanes, -1))
            qprev = q.at[jnp.maximum(lanes - 1, 0)].get(
                mode="promise_in_bounds")
            pe = jnp.where(lanes == 0, -1, qprev)
            cprev = csum.at[jnp.maximum(pe, 0)].get(
                mode="promise_in_bounds")
            runsum = csum - jnp.where(pe >= 0, cprev, 0.0)
            plsc.addupdate_scatter(den, [ks], runsum, mask=ends)

        def chunk(ch, carry):
            pltpu.sync_copy(src_h.at[wid, ch], srcs)
            pltpu.sync_copy(dst_h.at[wid, ch], dsts)
            for b in range(NB - 1):
                d16 = dsts[pl.ds(b * L, L)]
                pltpu.async_copy(h_h.at[d16], rows[b], gsem[b])

            def outer(o, carry2):
                for b in range(NB):
                    m = o * NB + b
                    process(m, b)
                    mn = m + NB - 1
                    bn = (b + NB - 1) % NB
                    @pl.when(jnp.logical_and(m >= 1, mn < GPC))
                    def _wait():
                        pltpu.make_async_copy(
                            rows[bn], acc.at[idxb.at[bn]], ssem[bn]).wait()

                    @pl.when(mn < GPC)
                    def _issue():
                        dn16 = dsts[pl.ds(mn * L, L)]
                        pltpu.async_copy(h_h.at[dn16], rows[bn], gsem[bn])
                return carry2

            lax.fori_loop(0, GPC // NB, outer, 0)
            for b in range(NB):
                pltpu.make_async_copy(rows[b], acc.at[idxb.at[b]], ssem[b]).wait()
            return carry

        lax.fori_loop(0, NCH, chunk, 0)
        pltpu.sync_copy(den, den_h.at[wid])
        plsc.subcore_barrier()
        pltpu.sync_copy(acc.at[pl.ds(sid * RPT, RPT)],
                        num_h.at[cid, pl.ds(sid * RPT, RPT)])

    return k(src3, dst3, s, t, h)


def _finish(num_parts, den_parts):
    B = 2000

    def body(n_ref, d_ref, o_ref):
        nblk = n_ref[...]
        num = nblk[0] + nblk[1]
        den = jnp.sum(d_ref[...], axis=1)
        safe = jnp.where(den == 0.0, 1.0, den)
        r = num / safe[:, None]
        neg = jnp.exp(jnp.minimum(r, 0.0)) - 1.0
        out = jnp.where(r > 0.0, r, neg)
        o_ref[...] = jnp.where(den[:, None] == 0.0, 0.0, out)

    return pl.pallas_call(
        body,
        grid=(N // B,),
        in_specs=[
            pl.BlockSpec((NC, B, D), lambda i: (0, i, 0)),
            pl.BlockSpec((B, NW), lambda i: (i, 0)),
        ],
        out_specs=pl.BlockSpec((B, D), lambda i: (i, 0)),
        out_shape=jax.ShapeDtypeStruct((N, D), jnp.float32),
    )(num_parts, den_parts)


def kernel(input, edge_list, W, a):
    src3 = edge_list[0].reshape(NW, NCH, CH)
    dst3 = edge_list[1].reshape(NW, NCH, CH)
    a2d = a.reshape(2, D)
    h, s, t = _prep(input, W, a2d)
    num_parts, den_parts = _sc_edge(src3, dst3, s, t, h)
    return _finish(num_parts, den_parts.T)


# speedup vs baseline: 32.7558x; 1.0003x over previous
import functools

import jax
import jax.numpy as jnp
from jax import lax
from jax.experimental import pallas as pl
from jax.experimental.pallas import tpu as pltpu
from jax.experimental.pallas import tpu_sc as plsc

N = 10000
E = 320000
D = 128

NC = 2
NS = 16
L = 16
NW = NC * NS
EPW = E // NW
G = EPW // L
DC = D + L
NP = 10240
RPT = NP // NS
CH = 2000
NCH = EPW // CH
GPC = CH // L
NB = 5


def _prep(x, w, a2d):
    B = 2000
    grid = N // B

    def body(x_ref, w_ref, a_ref, h_ref, s_ref, t_ref):
        h = jnp.dot(x_ref[...], w_ref[...], preferred_element_type=jnp.float32)
        h_ref[...] = h
        a1 = a_ref[0, :]
        a2 = a_ref[1, :]
        s_ref[...] = jnp.dot(h, a1, preferred_element_type=jnp.float32).reshape(1, 1, B)
        t_ref[...] = jnp.dot(h, a2, preferred_element_type=jnp.float32).reshape(1, 1, B)

    h, s3, t3 = pl.pallas_call(
        body,
        grid=(grid,),
        in_specs=[
            pl.BlockSpec((B, D), lambda i: (i, 0)),
            pl.BlockSpec((D, D), lambda i: (0, 0)),
            pl.BlockSpec((2, D), lambda i: (0, 0)),
        ],
        out_specs=[
            pl.BlockSpec((B, D), lambda i: (i, 0)),
            pl.BlockSpec((1, 1, B), lambda i: (i, 0, 0)),
            pl.BlockSpec((1, 1, B), lambda i: (i, 0, 0)),
        ],
        out_shape=[
            jax.ShapeDtypeStruct((N, D), jnp.float32),
            jax.ShapeDtypeStruct((grid, 1, B), jnp.float32),
            jax.ShapeDtypeStruct((grid, 1, B), jnp.float32),
        ],
    )(x, w, a2d)
    return h, s3.reshape(N), t3.reshape(N)


def _sc_edge(src3, dst3, s, t, h):
    mesh = plsc.VectorSubcoreMesh(core_axis_name="c", subcore_axis_name="s")

    @functools.partial(
        pl.kernel,
        out_type=(
            jax.ShapeDtypeStruct((NC, NP, D), jnp.float32),
            jax.ShapeDtypeStruct((NW, NP), jnp.float32),
        ),
        mesh=mesh,
        scratch_types=[
            pltpu.VMEM((CH,), jnp.int32),
            pltpu.VMEM((CH,), jnp.int32),
            pltpu.VMEM((N,), jnp.float32),
            pltpu.VMEM((N,), jnp.float32),
            pltpu.VMEM((NP,), jnp.float32),
            [pltpu.VMEM((L, D), jnp.float32) for _ in range(NB)],
            pltpu.VMEM((8, L), jnp.int32),
            pltpu.VMEM_SHARED((NP, D), jnp.float32),
            [pltpu.SemaphoreType.DMA for _ in range(NB)],
            [pltpu.SemaphoreType.DMA for _ in range(NB)],
        ],
        compiler_params=pltpu.CompilerParams(needs_layout_passes=False),
    )
    def k(src_h, dst_h, s_h, t_h, h_h, num_h, den_h,
          srcs, dsts, sv, tv, den, rows, idxb, acc, gsem, ssem):
        cid = lax.axis_index("c")
        sid = lax.axis_index("s")
        wid = sid * NC + cid

        pltpu.sync_copy(s_h, sv)
        pltpu.sync_copy(t_h, tv)

        zeros16 = jnp.zeros((L,), jnp.float32)

        def zden(i, carry):
            den[pl.ds(i * L, L)] = zeros16
            return carry

        lax.fori_loop(0, NP // L, zden, 0)
        for c in range(D // L):
            for r in range(L):
                rows[0][r, pl.ds(c * L, L)] = zeros16

        def zacc(i, carry):
            pltpu.sync_copy(rows[0], acc.at[pl.ds(sid * RPT + i * L, L)])
            return carry

        lax.fori_loop(0, RPT // L, zacc, 0)
        plsc.subcore_barrier()

        lanes = lax.iota(jnp.int32, L)

        def process(m, b):
            src16 = srcs[pl.ds(m * L, L)]
            dst16 = dsts[pl.ds(m * L, L)]
            svv = plsc.load_gather(sv, [src16])
            tvv = plsc.load_gather(tv, [dst16])
            x = svv + tvv
            e = jnp.where(x > 0.0, x, 0.2 * x)
            p = jnp.exp(e)
            pltpu.make_async_copy(h_h.at[dst16], rows[b], gsem[b]).wait()
            for r in range(L):
                pr = p[r]
                for c in range(D // L):
                    rows[b][r, pl.ds(c * L, L)] = rows[b][r, pl.ds(c * L, L)] * pr
            idxb[b, :] = src16
            pltpu.async_copy(rows[b], acc.at[idxb.at[b]], ssem[b], add=True)
            ks, vs = plsc.sort_key_val(src16, p)
            knext = ks.at[jnp.minimum(lanes + 1, L - 1)].get(
                mode="promise_in_bounds")
            ends = (ks != knext) | (lanes == L - 1)
            csum = plsc.cumsum(vs)
            q = plsc.cummax(jnp.where(ends, lanes, -1))
            qprev = q.at[jnp.maximum(lanes - 1, 0)].get(
                mode="promise_in_bounds")
            pe = jnp.where(lanes == 0, -1, qprev)
            cprev = csum.at[jnp.maximum(pe, 0)].get(
                mode="promise_in_bounds")
            runsum = csum - jnp.where(pe >= 0, cprev, 0.0)
            plsc.addupdate_scatter(den, [ks], runsum, mask=ends)

        def chunk(ch, carry):
            pltpu.sync_copy(src_h.at[wid, ch], srcs)
            pltpu.sync_copy(dst_h.at[wid, ch], dsts)
            for b in range(NB - 1):
                d16 = dsts[pl.ds(b * L, L)]
                pltpu.async_copy(h_h.at[d16], rows[b], gsem[b])

            def outer(o, carry2):
                for b in range(NB):
                    m = o * NB + b
                    process(m, b)
                    mn = m + NB - 1
                    bn = (b + NB - 1) % NB
                    @pl.when(jnp.logical_and(m >= 1, mn < GPC))
                    def _wait():
                        pltpu.make_async_copy(
                            rows[bn], acc.at[idxb.at[bn]], ssem[bn]).wait()

                    @pl.when(mn < GPC)
                    def _issue():
                        dn16 = dsts[pl.ds(mn * L, L)]
                        pltpu.async_copy(h_h.at[dn16], rows[bn], gsem[bn])
                return carry2

            lax.fori_loop(0, GPC // NB, outer, 0)
            for b in range(NB):
                pltpu.make_async_copy(rows[b], acc.at[idxb.at[b]], ssem[b]).wait()
            return carry

        lax.fori_loop(0, NCH, chunk, 0)
        pltpu.sync_copy(den, den_h.at[wid])
        plsc.subcore_barrier()
        pltpu.sync_copy(acc.at[pl.ds(sid * RPT, RPT)],
                        num_h.at[cid, pl.ds(sid * RPT, RPT)])

    return k(src3, dst3, s, t, h)


def _finish(num_parts, den_parts):
    B = 2048

    def body(n_ref, d_ref, o_ref):
        nblk = n_ref[...]
        num = nblk[0] + nblk[1]
        den = jnp.sum(d_ref[...], axis=0)
        safe = jnp.where(den == 0.0, 1.0, den)
        r = num / safe[:, None]
        neg = jnp.exp(jnp.minimum(r, 0.0)) - 1.0
        out = jnp.where(r > 0.0, r, neg)
        o_ref[...] = jnp.where(den[:, None] == 0.0, 0.0, out)

    return pl.pallas_call(
        body,
        grid=(NP // B,),
        in_specs=[
            pl.BlockSpec((NC, B, D), lambda i: (0, i, 0)),
            pl.BlockSpec((NW, B), lambda i: (0, i)),
        ],
        out_specs=pl.BlockSpec((B, D), lambda i: (i, 0)),
        out_shape=jax.ShapeDtypeStruct((NP, D), jnp.float32),
    )(num_parts, den_parts)


def kernel(input, edge_list, W, a):
    src3 = edge_list[0].reshape(NW, NCH, CH)
    dst3 = edge_list[1].reshape(NW, NCH, CH)
    a2d = a.reshape(2, D)
    h, s, t = _prep(input, W, a2d)
    num_parts, den_parts = _sc_edge(src3, dst3, s, t, h)
    return _finish(num_parts, den_parts)[:N]


# speedup vs baseline: 32.7627x; 1.0002x over previous
import functools

import jax
import jax.numpy as jnp
from jax import lax
from jax.experimental import pallas as pl
from jax.experimental.pallas import tpu as pltpu
from jax.experimental.pallas import tpu_sc as plsc

N = 10000
E = 320000
D = 128

NC = 2
NS = 16
L = 16
NW = NC * NS
EPW = E // NW
G = EPW // L
DC = D + L
NP = 10240
RPT = NP // NS
CH = 2000
NCH = EPW // CH
GPC = CH // L
NB = 5


def _prep(x, w, a2d):
    B = 2000
    grid = N // B

    def body(x_ref, w_ref, a_ref, h_ref, s_ref, t_ref):
        h = jnp.dot(x_ref[...], w_ref[...], preferred_element_type=jnp.float32)
        h_ref[...] = h
        a1 = a_ref[0, :]
        a2 = a_ref[1, :]
        s_ref[...] = jnp.dot(h, a1, preferred_element_type=jnp.float32).reshape(1, 1, B)
        t_ref[...] = jnp.dot(h, a2, preferred_element_type=jnp.float32).reshape(1, 1, B)

    h, s3, t3 = pl.pallas_call(
        body,
        grid=(grid,),
        in_specs=[
            pl.BlockSpec((B, D), lambda i: (i, 0)),
            pl.BlockSpec((D, D), lambda i: (0, 0)),
            pl.BlockSpec((2, D), lambda i: (0, 0)),
        ],
        out_specs=[
            pl.BlockSpec((B, D), lambda i: (i, 0)),
            pl.BlockSpec((1, 1, B), lambda i: (i, 0, 0)),
            pl.BlockSpec((1, 1, B), lambda i: (i, 0, 0)),
        ],
        out_shape=[
            jax.ShapeDtypeStruct((N, D), jnp.float32),
            jax.ShapeDtypeStruct((grid, 1, B), jnp.float32),
            jax.ShapeDtypeStruct((grid, 1, B), jnp.float32),
        ],
    )(x, w, a2d)
    return h, s3.reshape(N), t3.reshape(N)


def _sc_edge(src3, dst3, s, t, h):
    mesh = plsc.VectorSubcoreMesh(core_axis_name="c", subcore_axis_name="s")

    @functools.partial(
        pl.kernel,
        out_type=(
            jax.ShapeDtypeStruct((NC, NP, D), jnp.float32),
            jax.ShapeDtypeStruct((NW, NP), jnp.float32),
        ),
        mesh=mesh,
        scratch_types=[
            pltpu.VMEM((CH,), jnp.int32),
            pltpu.VMEM((CH,), jnp.int32),
            pltpu.VMEM((N,), jnp.float32),
            pltpu.VMEM((N,), jnp.float32),
            pltpu.VMEM((NP,), jnp.float32),
            [pltpu.VMEM((L, D), jnp.float32) for _ in range(NB)],
            pltpu.VMEM((8, L), jnp.int32),
            pltpu.VMEM_SHARED((NP, D), jnp.float32),
            [pltpu.SemaphoreType.DMA for _ in range(NB)],
            [pltpu.SemaphoreType.DMA for _ in range(NB)],
        ],
        compiler_params=pltpu.CompilerParams(needs_layout_passes=False),
    )
    def k(src_h, dst_h, s_h, t_h, h_h, num_h, den_h,
          srcs, dsts, sv, tv, den, rows, idxb, acc, gsem, ssem):
        cid = lax.axis_index("c")
        sid = lax.axis_index("s")
        wid = sid * NC + cid

        pltpu.sync_copy(s_h, sv)
        pltpu.sync_copy(t_h, tv)

        zeros16 = jnp.zeros((L,), jnp.float32)

        def zden(i, carry):
            den[pl.ds(i * L, L)] = zeros16
            return carry

        lax.fori_loop(0, NP // L, zden, 0)
        for c in range(D // L):
            for r in range(L):
                rows[0][r, pl.ds(c * L, L)] = zeros16

        def zacc(i, carry):
            pltpu.sync_copy(rows[0], acc.at[pl.ds(sid * RPT + i * L, L)])
            return carry

        lax.fori_loop(0, RPT // L, zacc, 0)
        plsc.subcore_barrier()

        lanes = lax.iota(jnp.int32, L)

        def process(m, b):
            src16 = srcs[pl.ds(m * L, L)]
            dst16 = dsts[pl.ds(m * L, L)]
            svv = plsc.load_gather(sv, [src16])
            tvv = plsc.load_gather(tv, [dst16])
            x = svv + tvv
            e = jnp.where(x > 0.0, x, 0.2 * x)
            p = jnp.exp(e)
            pltpu.make_async_copy(h_h.at[dst16], rows[b], gsem[b]).wait()
            for r in range(L):
                pr = p[r]
                for c in range(D // L):
                    rows[b][r, pl.ds(c * L, L)] = rows[b][r, pl.ds(c * L, L)] * pr
            idxb[b, :] = src16
            pltpu.async_copy(rows[b], acc.at[idxb.at[b]], ssem[b], add=True)
            plsc.addupdate_scatter(den, [src16], p)

        def chunk(ch, carry):
            pltpu.sync_copy(src_h.at[wid, ch], srcs)
            pltpu.sync_copy(dst_h.at[wid, ch], dsts)
            for b in range(NB - 1):
                d16 = dsts[pl.ds(b * L, L)]
                pltpu.async_copy(h_h.at[d16], rows[b], gsem[b])

            def outer(o, carry2):
                for b in range(NB):
                    m = o * NB + b
                    process(m, b)
                    mn = m + NB - 1
                    bn = (b + NB - 1) % NB
                    @pl.when(jnp.logical_and(m >= 1, mn < GPC))
                    def _wait():
                        pltpu.make_async_copy(
                            rows[bn], acc.at[idxb.at[bn]], ssem[bn]).wait()

                    @pl.when(mn < GPC)
                    def _issue():
                        dn16 = dsts[pl.ds(mn * L, L)]
                        pltpu.async_copy(h_h.at[dn16], rows[bn], gsem[bn])
                return carry2

            lax.fori_loop(0, GPC // NB, outer, 0)
            for b in range(NB):
                pltpu.make_async_copy(rows[b], acc.at[idxb.at[b]], ssem[b]).wait()
            return carry

        lax.fori_loop(0, NCH, chunk, 0)
        pltpu.sync_copy(den, den_h.at[wid])
        plsc.subcore_barrier()
        pltpu.sync_copy(acc.at[pl.ds(sid * RPT, RPT)],
                        num_h.at[cid, pl.ds(sid * RPT, RPT)])

    return k(src3, dst3, s, t, h)


def _finish(num_parts, den_parts):
    B = 2048

    def body(n_ref, d_ref, o_ref):
        nblk = n_ref[...]
        num = nblk[0] + nblk[1]
        den = jnp.sum(d_ref[...], axis=0)
        safe = jnp.where(den == 0.0, 1.0, den)
        r = num / safe[:, None]
        neg = jnp.exp(jnp.minimum(r, 0.0)) - 1.0
        out = jnp.where(r > 0.0, r, neg)
        o_ref[...] = jnp.where(den[:, None] == 0.0, 0.0, out)

    return pl.pallas_call(
        body,
        grid=(NP // B,),
        in_specs=[
            pl.BlockSpec((NC, B, D), lambda i: (0, i, 0)),
            pl.BlockSpec((NW, B), lambda i: (0, i)),
        ],
        out_specs=pl.BlockSpec((B, D), lambda i: (i, 0)),
        out_shape=jax.ShapeDtypeStruct((NP, D), jnp.float32),
    )(num_parts, den_parts)


def kernel(input, edge_list, W, a):
    src3 = edge_list[0].reshape(NW, NCH, CH)
    dst3 = edge_list[1].reshape(NW, NCH, CH)
    a2d = a.reshape(2, D)
    h, s, t = _prep(input, W, a2d)
    num_parts, den_parts = _sc_edge(src3, dst3, s, t, h)
    return _finish(num_parts, den_parts)[:N]
